# baseline (speedup 1.0000x reference)
import functools

import jax
import jax.numpy as jnp
from jax import lax
from jax.experimental import pallas as pl
from jax.experimental.pallas import tpu as pltpu
from jax.experimental.pallas import tpu_sc as plsc

_PREC = lax.Precision.DEFAULT
_K = 5


def _tables_body(x_ref, ws_ref, wr_ref, ts_ref, tr_ref):
    xb = x_ref[...]
    ts_ref[...] = jnp.dot(xb, ws_ref[...], preferred_element_type=jnp.float32,
                          precision=_PREC)
    tr_ref[...] = jnp.dot(xb, wr_ref[...], preferred_element_type=jnp.float32,
                          precision=_PREC)


def _make_tables(x, ws, wr):
    n, h = x.shape
    tb = 2000
    return pl.pallas_call(
        _tables_body,
        grid=(n // tb,),
        in_specs=[
            pl.BlockSpec((tb, h), lambda i: (i, 0)),
            pl.BlockSpec((h, h), lambda i: (0, 0)),
            pl.BlockSpec((h, h), lambda i: (0, 0)),
        ],
        out_specs=[
            pl.BlockSpec((tb, h), lambda i: (i, 0)),
            pl.BlockSpec((tb, h), lambda i: (i, 0)),
        ],
        out_shape=[jax.ShapeDtypeStruct((n, h), jnp.float32)] * 2,
    )(x, ws, wr)


_GW = 80


def _sc_gather(table, idx):
    n, h = table.shape
    e = idx.shape[0]
    mesh = plsc.VectorSubcoreMesh(core_axis_name="core",
                                  subcore_axis_name="subcore")

    @functools.partial(
        pl.kernel,
        out_type=jax.ShapeDtypeStruct((e, h), jnp.float32),
        mesh=mesh,
    )
    def k(t_hbm, i_hbm, o_hbm):
        def body(i_vmem, o_vmem):
            pltpu.sync_copy(t_hbm.at[i_vmem.at[0, 0]], o_vmem)

        pltpu.emit_pipeline(
            body,
            grid=(e // _GW,),
            in_specs=[pl.BlockSpec((1, 1, _GW), lambda i: (i, 0, 0))],
            out_specs=[pl.BlockSpec((_GW, h), lambda i: (i, 0))],
            core_axis_name=("core", "subcore"),
            dimension_semantics=(pltpu.PARALLEL,),
        )(i_hbm, o_hbm)

    return k(table, idx.reshape(e // _GW, 1, _GW))


_NPAD = 10240
_OP = 12000


def _sc_scatter(en, receivers):
    e, h = en.shape
    n_sub = 16
    rows_per_sub = _NPAD // n_sub
    zb = 128
    mesh = plsc.VectorSubcoreMesh(core_axis_name="core",
                                  subcore_axis_name="subcore")

    @functools.partial(
        pl.kernel,
        out_type=jax.ShapeDtypeStruct((2 * _OP, h), jnp.float32),
        mesh=mesh,
        scratch_types=[
            pltpu.VMEM((zb, h), jnp.float32),
            pltpu.VMEM_SHARED((_NPAD, h), jnp.float32),
        ],
    )
    def k(en_hbm, r_hbm, out_hbm, zbuf, agg_sh):
        cid = lax.axis_index("core")
        sid = lax.axis_index("subcore")

        @pl.loop(0, zb)
        def _(rr):
            for j in range(h // 16):
                zbuf.at[pl.ds(rr, 1), pl.ds(j * 16, 16)][...] = (
                    jnp.zeros((1, 16), jnp.float32))

        @pl.loop(0, rows_per_sub // zb)
        def _(kk):
            pltpu.sync_copy(
                zbuf, agg_sh.at[pl.ds(sid * rows_per_sub + kk * zb, zb)])

        plsc.subcore_barrier()

        def body(en_vmem, r_vmem):
            pltpu.sync_copy(en_vmem, agg_sh.at[r_vmem.at[0, 0]], add=True)

        pltpu.emit_pipeline(
            body,
            grid=(e // _GW,),
            in_specs=[pl.BlockSpec((_GW, h), lambda i: (i, 0)),
                      pl.BlockSpec((1, 1, _GW), lambda i: (i, 0, 0))],
            out_specs=[],
            core_axis_name=("core", "subcore"),
            dimension_semantics=(pltpu.PARALLEL,),
        )(en_hbm, r_hbm)

        plsc.subcore_barrier()

        @pl.loop(0, rows_per_sub // zb)
        def _(kk):
            pltpu.sync_copy(
                agg_sh.at[pl.ds(sid * rows_per_sub + kk * zb, zb)], zbuf)
            pltpu.sync_copy(
                zbuf,
                out_hbm.at[
                    pl.ds(cid * _OP + sid * rows_per_sub + kk * zb, zb)])

    return k(en, receivers.reshape(e // _GW, 1, _GW))


def _edge_body(gs_ref, gr_ref, attr_ref, eo_in_ref, w0e, b0, w1, b1, w2, b2,
               w3, b3, g, beta, en_ref, eo_ref):
    del eo_in_ref
    attr = attr_ref[...]
    h = (gs_ref[...] + gr_ref[...] + b0[...]
         + jnp.dot(attr, w0e[...], preferred_element_type=jnp.float32,
                   precision=_PREC))
    h = jnp.maximum(h, 0.0)
    h = jnp.maximum(
        jnp.dot(h, w1[...], preferred_element_type=jnp.float32,
                precision=_PREC) + b1[...], 0.0)
    h = jnp.maximum(
        jnp.dot(h, w2[...], preferred_element_type=jnp.float32,
                precision=_PREC) + b2[...], 0.0)
    h = jnp.dot(h, w3[...], preferred_element_type=jnp.float32,
                precision=_PREC) + b3[...]
    mu = jnp.mean(h, axis=-1, keepdims=True)
    d = h - mu
    var = jnp.mean(d * d, axis=-1, keepdims=True)
    en = (d * lax.rsqrt(var + 1e-5)) * g[...] + beta[...]
    en_ref[...] = en
    eo_ref[...] = attr + en


def _edge_mlp_chunk(gs_c, gr_c, attr, eo_buf, c, w0e, b0, w1, b1, w2, b2, w3,
                    b3, g, beta):
    e, h = attr.shape
    ec = gs_c.shape[0]
    te = 8000
    steps = ec // te
    off = c * steps
    row = lambda i: (i, 0)
    offrow = lambda i: (i + off, 0)
    whole = lambda i: (0, 0)
    wspec = pl.BlockSpec((h, h), whole)
    bspec = pl.BlockSpec((1, h), whole)
    return pl.pallas_call(
        _edge_body,
        grid=(steps,),
        in_specs=[pl.BlockSpec((te, h), row)] * 2
        + [pl.BlockSpec((te, h), offrow),
           pl.BlockSpec(memory_space=pl.ANY)]
        + [wspec, bspec, wspec, bspec, wspec, bspec, wspec, bspec,
           bspec, bspec],
        out_specs=[pl.BlockSpec((te, h), row),
                   pl.BlockSpec((te, h), offrow)],
        out_shape=[jax.ShapeDtypeStruct((ec, h), jnp.float32),
                   jax.ShapeDtypeStruct((e, h), jnp.float32)],
        input_output_aliases={3: 1},
    )(gs_c, gr_c, attr, eo_buf, w0e, b0, w1, b1, w2, b2, w3, b3, g, beta)


def _node_body(x_ref, p0, p1, p2, p3, p4, p5, p6, p7, p8, p9, wx, wa, b0,
               w1, b1, w2, b2, w3, b3, g, beta, xo_ref):
    xb = x_ref[...]
    agg = ((p0[...] + p1[...]) + (p2[...] + p3[...])
           + (p4[...] + p5[...]) + (p6[...] + p7[...])
           + (p8[...] + p9[...]))
    h = (jnp.dot(xb, wx[...], preferred_element_type=jnp.float32,
                 precision=_PREC)
         + jnp.dot(agg, wa[...], preferred_element_type=jnp.float32,
                   precision=_PREC) + b0[...])
    h = jnp.maximum(h, 0.0)
    h = jnp.maximum(
        jnp.dot(h, w1[...], preferred_element_type=jnp.float32,
                precision=_PREC) + b1[...], 0.0)
    h = jnp.maximum(
        jnp.dot(h, w2[...], preferred_element_type=jnp.float32,
                precision=_PREC) + b2[...], 0.0)
    h = jnp.dot(h, w3[...], preferred_element_type=jnp.float32,
                precision=_PREC) + b3[...]
    mu = jnp.mean(h, axis=-1, keepdims=True)
    d = h - mu
    var = jnp.mean(d * d, axis=-1, keepdims=True)
    xo_ref[...] = xb + (d * lax.rsqrt(var + 1e-5)) * g[...] + beta[...]


def _node_mlp(x, parts, wx, wa, b0, w1, b1, w2, b2, w3, b3, g, beta):
    n, h = x.shape
    tn = 2000
    row = lambda i: (i, 0)
    row1 = lambda i: (i + _OP // 2000, 0)
    whole = lambda i: (0, 0)
    wspec = pl.BlockSpec((h, h), whole)
    bspec = pl.BlockSpec((1, h), whole)
    pspecs = []
    pargs = []
    for p in parts:
        pspecs += [pl.BlockSpec((tn, h), row), pl.BlockSpec((tn, h), row1)]
        pargs += [p, p]
    return pl.pallas_call(
        _node_body,
        grid=(n // tn,),
        in_specs=[pl.BlockSpec((tn, h), row)] + pspecs
        + [wspec, wspec, bspec, wspec, bspec, wspec, bspec, wspec, bspec,
           bspec, bspec],
        out_specs=pl.BlockSpec((tn, h), row),
        out_shape=jax.ShapeDtypeStruct((n, h), jnp.float32),
    )(x, *pargs, wx, wa, b0, w1, b1, w2, b2, w3, b3, g, beta)


def kernel(x, edge_index, edge_attr, eb_W0, eb_b0, eb_W1, eb_b1, eb_W2, eb_b2,
           eb_W3, eb_b3, eb_g, eb_beta, nb_W0, nb_b0, nb_W1, nb_b1, nb_W2,
           nb_b2, nb_W3, nb_b3, nb_g, nb_beta):
    n, h = x.shape
    e = edge_attr.shape[0]
    ec = e // _K
    senders = edge_index[0].reshape(_K, ec)
    receivers = edge_index[1].reshape(_K, ec)

    r2 = lambda v: v.reshape(1, h)
    eb = (eb_W0[2 * h:], r2(eb_b0), eb_W1, r2(eb_b1), eb_W2, r2(eb_b2),
          eb_W3, r2(eb_b3), r2(eb_g), r2(eb_beta))

    ts, tr = _make_tables(x, eb_W0[:h], eb_W0[h:2 * h])

    eo_buf = jnp.zeros((e, h), jnp.float32)
    parts = []
    for c in range(_K):
        gs_c = _sc_gather(ts, senders[c])
        gr_c = _sc_gather(tr, receivers[c])
        en_c, eo_buf = _edge_mlp_chunk(gs_c, gr_c, edge_attr, eo_buf, c, *eb)
        parts.append(_sc_scatter(en_c, receivers[c]))

    xo = _node_mlp(x, parts, nb_W0[:h], nb_W0[h:], r2(nb_b0), nb_W1,
                   r2(nb_b1), nb_W2, r2(nb_b2), nb_W3, r2(nb_b3), r2(nb_g),
                   r2(nb_beta))
    return (xo, eo_buf)


# speedup vs baseline: 1.0250x; 1.0250x over previous
import functools

import jax
import jax.numpy as jnp
from jax import lax
from jax.experimental import pallas as pl
from jax.experimental.pallas import tpu as pltpu
from jax.experimental.pallas import tpu_sc as plsc

_PREC = lax.Precision.DEFAULT
_K = 5


def _tables_body(x_ref, ws_ref, wr_ref, ts_ref, tr_ref):
    xb = x_ref[...]
    ts_ref[...] = jnp.dot(xb, ws_ref[...], preferred_element_type=jnp.float32,
                          precision=_PREC)
    tr_ref[...] = jnp.dot(xb, wr_ref[...], preferred_element_type=jnp.float32,
                          precision=_PREC)


def _make_tables(x, ws, wr):
    n, h = x.shape
    tb = 2000
    return pl.pallas_call(
        _tables_body,
        grid=(n // tb,),
        in_specs=[
            pl.BlockSpec((tb, h), lambda i: (i, 0)),
            pl.BlockSpec((h, h), lambda i: (0, 0)),
            pl.BlockSpec((h, h), lambda i: (0, 0)),
        ],
        out_specs=[
            pl.BlockSpec((tb, h), lambda i: (i, 0)),
            pl.BlockSpec((tb, h), lambda i: (i, 0)),
        ],
        out_shape=[jax.ShapeDtypeStruct((n, h), jnp.float32)] * 2,
    )(x, ws, wr)


_GW = 80


def _sc_gather(table, idx):
    n, h = table.shape
    e = idx.shape[0]
    mesh = plsc.VectorSubcoreMesh(core_axis_name="core",
                                  subcore_axis_name="subcore")

    @functools.partial(
        pl.kernel,
        out_type=jax.ShapeDtypeStruct((e, h), table.dtype),
        mesh=mesh,
    )
    def k(t_hbm, i_hbm, o_hbm):
        def body(i_vmem, o_vmem):
            pltpu.sync_copy(t_hbm.at[i_vmem.at[0, 0]], o_vmem)

        pltpu.emit_pipeline(
            body,
            grid=(e // _GW,),
            in_specs=[pl.BlockSpec((1, 1, _GW), lambda i: (i, 0, 0))],
            out_specs=[pl.BlockSpec((_GW, h), lambda i: (i, 0))],
            core_axis_name=("core", "subcore"),
            dimension_semantics=(pltpu.PARALLEL,),
        )(i_hbm, o_hbm)

    return k(table, idx.reshape(e // _GW, 1, _GW))


_NPAD = 10240
_OP = 12000


def _sc_scatter(en, receivers):
    e, h = en.shape
    n_sub = 16
    rows_per_sub = _NPAD // n_sub
    zb = 128
    mesh = plsc.VectorSubcoreMesh(core_axis_name="core",
                                  subcore_axis_name="subcore")

    @functools.partial(
        pl.kernel,
        out_type=jax.ShapeDtypeStruct((2 * _OP, h), jnp.float32),
        mesh=mesh,
        scratch_types=[
            pltpu.VMEM((zb, h), jnp.float32),
            pltpu.VMEM_SHARED((_NPAD, h), jnp.float32),
        ],
    )
    def k(en_hbm, r_hbm, out_hbm, zbuf, agg_sh):
        cid = lax.axis_index("core")
        sid = lax.axis_index("subcore")

        @pl.loop(0, zb)
        def _(rr):
            for j in range(h // 16):
                zbuf.at[pl.ds(rr, 1), pl.ds(j * 16, 16)][...] = (
                    jnp.zeros((1, 16), jnp.float32))

        @pl.loop(0, rows_per_sub // zb)
        def _(kk):
            pltpu.sync_copy(
                zbuf, agg_sh.at[pl.ds(sid * rows_per_sub + kk * zb, zb)])

        plsc.subcore_barrier()

        def body(en_vmem, r_vmem):
            pltpu.sync_copy(en_vmem, agg_sh.at[r_vmem.at[0, 0]], add=True)

        pltpu.emit_pipeline(
            body,
            grid=(e // _GW,),
            in_specs=[pl.BlockSpec((_GW, h), lambda i: (i, 0)),
                      pl.BlockSpec((1, 1, _GW), lambda i: (i, 0, 0))],
            out_specs=[],
            core_axis_name=("core", "subcore"),
            dimension_semantics=(pltpu.PARALLEL,),
        )(en_hbm, r_hbm)

        plsc.subcore_barrier()

        @pl.loop(0, rows_per_sub // zb)
        def _(kk):
            pltpu.sync_copy(
                agg_sh.at[pl.ds(sid * rows_per_sub + kk * zb, zb)], zbuf)
            pltpu.sync_copy(
                zbuf,
                out_hbm.at[
                    pl.ds(cid * _OP + sid * rows_per_sub + kk * zb, zb)])

    return k(en, receivers.reshape(e // _GW, 1, _GW))


def _edge_body(gs_ref, gr_ref, attr_ref, eo_in_ref, w0e, b0, w1, b1, w2, b2,
               w3, b3, g, beta, en_ref, eo_ref):
    del eo_in_ref
    attr = attr_ref[...]
    h = (gs_ref[...] + gr_ref[...] + b0[...]
         + jnp.dot(attr, w0e[...], preferred_element_type=jnp.float32,
                   precision=_PREC))
    h = jnp.maximum(h, 0.0)
    h = jnp.maximum(
        jnp.dot(h, w1[...], preferred_element_type=jnp.float32,
                precision=_PREC) + b1[...], 0.0)
    h = jnp.maximum(
        jnp.dot(h, w2[...], preferred_element_type=jnp.float32,
                precision=_PREC) + b2[...], 0.0)
    h = jnp.dot(h, w3[...], preferred_element_type=jnp.float32,
                precision=_PREC) + b3[...]
    mu = jnp.mean(h, axis=-1, keepdims=True)
    d = h - mu
    var = jnp.mean(d * d, axis=-1, keepdims=True)
    en = (d * lax.rsqrt(var + 1e-5)) * g[...] + beta[...]
    en_ref[...] = en
    eo_ref[...] = attr + en


def _edge_body0(gs_ref, gr_ref, attr_ref, w0e, b0, w1, b1, w2, b2,
                w3, b3, g, beta, en_ref, eo_ref):
    _edge_body(gs_ref, gr_ref, attr_ref, None, w0e, b0, w1, b1, w2, b2,
               w3, b3, g, beta, en_ref, eo_ref)


def _edge_mlp_chunk(gs_c, gr_c, attr, eo_buf, c, w0e, b0, w1, b1, w2, b2, w3,
                    b3, g, beta):
    e, h = attr.shape
    ec = gs_c.shape[0]
    te = 8000
    steps = ec // te
    off = c * steps
    row = lambda i: (i, 0)
    offrow = lambda i: (i + off, 0)
    whole = lambda i: (0, 0)
    wspec = pl.BlockSpec((h, h), whole)
    bspec = pl.BlockSpec((1, h), whole)
    gspecs = [pl.BlockSpec((te, h), row)] * 2 + [pl.BlockSpec((te, h), offrow)]
    wspecs = [wspec, bspec, wspec, bspec, wspec, bspec, wspec, bspec,
              bspec, bspec]
    out_specs = [pl.BlockSpec((te, h), row), pl.BlockSpec((te, h), offrow)]
    out_shape = [jax.ShapeDtypeStruct((ec, h), jnp.float32),
                 jax.ShapeDtypeStruct((e, h), jnp.float32)]
    wargs = (w0e, b0, w1, b1, w2, b2, w3, b3, g, beta)
    if eo_buf is None:
        return pl.pallas_call(
            _edge_body0,
            grid=(steps,),
            in_specs=gspecs + wspecs,
            out_specs=out_specs,
            out_shape=out_shape,
        )(gs_c, gr_c, attr, *wargs)
    return pl.pallas_call(
        _edge_body,
        grid=(steps,),
        in_specs=gspecs + [pl.BlockSpec(memory_space=pl.ANY)] + wspecs,
        out_specs=out_specs,
        out_shape=out_shape,
        input_output_aliases={3: 1},
    )(gs_c, gr_c, attr, eo_buf, *wargs)


def _node_body(x_ref, p0, p1, p2, p3, p4, p5, p6, p7, p8, p9, wx, wa, b0,
               w1, b1, w2, b2, w3, b3, g, beta, xo_ref):
    xb = x_ref[...]
    agg = ((p0[...] + p1[...]) + (p2[...] + p3[...])
           + (p4[...] + p5[...]) + (p6[...] + p7[...])
           + (p8[...] + p9[...]))
    h = (jnp.dot(xb, wx[...], preferred_element_type=jnp.float32,
                 precision=_PREC)
         + jnp.dot(agg, wa[...], preferred_element_type=jnp.float32,
                   precision=_PREC) + b0[...])
    h = jnp.maximum(h, 0.0)
    h = jnp.maximum(
        jnp.dot(h, w1[...], preferred_element_type=jnp.float32,
                precision=_PREC) + b1[...], 0.0)
    h = jnp.maximum(
        jnp.dot(h, w2[...], preferred_element_type=jnp.float32,
                precision=_PREC) + b2[...], 0.0)
    h = jnp.dot(h, w3[...], preferred_element_type=jnp.float32,
                precision=_PREC) + b3[...]
    mu = jnp.mean(h, axis=-1, keepdims=True)
    d = h - mu
    var = jnp.mean(d * d, axis=-1, keepdims=True)
    xo_ref[...] = xb + (d * lax.rsqrt(var + 1e-5)) * g[...] + beta[...]


def _node_mlp(x, parts, wx, wa, b0, w1, b1, w2, b2, w3, b3, g, beta):
    n, h = x.shape
    tn = 2000
    row = lambda i: (i, 0)
    row1 = lambda i: (i + _OP // 2000, 0)
    whole = lambda i: (0, 0)
    wspec = pl.BlockSpec((h, h), whole)
    bspec = pl.BlockSpec((1, h), whole)
    pspecs = []
    pargs = []
    for p in parts:
        pspecs += [pl.BlockSpec((tn, h), row), pl.BlockSpec((tn, h), row1)]
        pargs += [p, p]
    return pl.pallas_call(
        _node_body,
        grid=(n // tn,),
        in_specs=[pl.BlockSpec((tn, h), row)] + pspecs
        + [wspec, wspec, bspec, wspec, bspec, wspec, bspec, wspec, bspec,
           bspec, bspec],
        out_specs=pl.BlockSpec((tn, h), row),
        out_shape=jax.ShapeDtypeStruct((n, h), jnp.float32),
    )(x, *pargs, wx, wa, b0, w1, b1, w2, b2, w3, b3, g, beta)


def kernel(x, edge_index, edge_attr, eb_W0, eb_b0, eb_W1, eb_b1, eb_W2, eb_b2,
           eb_W3, eb_b3, eb_g, eb_beta, nb_W0, nb_b0, nb_W1, nb_b1, nb_W2,
           nb_b2, nb_W3, nb_b3, nb_g, nb_beta):
    n, h = x.shape
    e = edge_attr.shape[0]
    ec = e // _K
    senders = edge_index[0].reshape(_K, ec)
    receivers = edge_index[1].reshape(_K, ec)

    r2 = lambda v: v.reshape(1, h)
    eb = (eb_W0[2 * h:], r2(eb_b0), eb_W1, r2(eb_b1), eb_W2, r2(eb_b2),
          eb_W3, r2(eb_b3), r2(eb_g), r2(eb_beta))

    ts, tr = _make_tables(x, eb_W0[:h], eb_W0[h:2 * h])

    eo_buf = None
    parts = []
    for c in range(_K):
        gs_c = _sc_gather(ts, senders[c])
        gr_c = _sc_gather(tr, receivers[c])
        en_c, eo_buf = _edge_mlp_chunk(gs_c, gr_c, edge_attr, eo_buf, c, *eb)
        parts.append(_sc_scatter(en_c, receivers[c]))

    xo = _node_mlp(x, parts, nb_W0[:h], nb_W0[h:], r2(nb_b0), nb_W1,
                   r2(nb_b1), nb_W2, r2(nb_b2), nb_W3, r2(nb_b3), r2(nb_g),
                   r2(nb_beta))
    return (xo, eo_buf)


# speedup vs baseline: 1.0495x; 1.0239x over previous
import functools

import jax
import jax.numpy as jnp
from jax import lax
from jax.experimental import pallas as pl
from jax.experimental.pallas import tpu as pltpu
from jax.experimental.pallas import tpu_sc as plsc

_PREC = lax.Precision.DEFAULT
_K = 5


def _pack_bf16_pair(t):
    m = t.shape[1] // 2
    a = jax.lax.bitcast_convert_type(t[:, :m], jnp.int32)
    b = jax.lax.bitcast_convert_type(t[:, m:], jnp.int32)
    a = jax.lax.shift_right_logical(a + 0x8000, 16)
    b = (b + 0x8000) & jnp.int32(-65536)
    return a | b


def _tables_body(x_ref, ws_ref, wr_ref, ts_ref, tr_ref):
    xb = x_ref[...]
    ts = jnp.dot(xb, ws_ref[...], preferred_element_type=jnp.float32,
                 precision=_PREC)
    tr = jnp.dot(xb, wr_ref[...], preferred_element_type=jnp.float32,
                 precision=_PREC)
    ts_ref[...] = _pack_bf16_pair(ts)
    tr_ref[...] = _pack_bf16_pair(tr)


def _make_tables(x, ws, wr):
    n, h = x.shape
    tb = 2000
    return pl.pallas_call(
        _tables_body,
        grid=(n // tb,),
        in_specs=[
            pl.BlockSpec((tb, h), lambda i: (i, 0)),
            pl.BlockSpec((h, h), lambda i: (0, 0)),
            pl.BlockSpec((h, h), lambda i: (0, 0)),
        ],
        out_specs=[
            pl.BlockSpec((tb, h // 2), lambda i: (i, 0)),
            pl.BlockSpec((tb, h // 2), lambda i: (i, 0)),
        ],
        out_shape=[jax.ShapeDtypeStruct((n, h // 2), jnp.int32)] * 2,
    )(x, ws, wr)


_GW = 80


def _sc_gather(table, idx):
    n, h = table.shape
    e = idx.shape[0]
    mesh = plsc.VectorSubcoreMesh(core_axis_name="core",
                                  subcore_axis_name="subcore")

    @functools.partial(
        pl.kernel,
        out_type=jax.ShapeDtypeStruct((e, h), table.dtype),
        mesh=mesh,
        compiler_params=pltpu.CompilerParams(use_tc_tiling_on_sc=False),
    )
    def k(t_hbm, i_hbm, o_hbm):
        def body(i_vmem, o_vmem):
            pltpu.sync_copy(t_hbm.at[i_vmem.at[0, 0]], o_vmem)

        pltpu.emit_pipeline(
            body,
            grid=(e // _GW,),
            in_specs=[pl.BlockSpec((1, 1, _GW), lambda i: (i, 0, 0))],
            out_specs=[pl.BlockSpec((_GW, h), lambda i: (i, 0))],
            core_axis_name=("core", "subcore"),
            dimension_semantics=(pltpu.PARALLEL,),
        )(i_hbm, o_hbm)

    return k(table, idx.reshape(e // _GW, 1, _GW))


_NPAD = 10240
_OP = 12000


def _sc_scatter(en, receivers):
    e, h = en.shape
    n_sub = 16
    rows_per_sub = _NPAD // n_sub
    zb = 128
    mesh = plsc.VectorSubcoreMesh(core_axis_name="core",
                                  subcore_axis_name="subcore")

    @functools.partial(
        pl.kernel,
        out_type=jax.ShapeDtypeStruct((2 * _OP, h), jnp.float32),
        mesh=mesh,
        scratch_types=[
            pltpu.VMEM((zb, h), jnp.float32),
            pltpu.VMEM_SHARED((_NPAD, h), jnp.float32),
        ],
    )
    def k(en_hbm, r_hbm, out_hbm, zbuf, agg_sh):
        cid = lax.axis_index("core")
        sid = lax.axis_index("subcore")

        @pl.loop(0, zb)
        def _(rr):
            for j in range(h // 16):
                zbuf.at[pl.ds(rr, 1), pl.ds(j * 16, 16)][...] = (
                    jnp.zeros((1, 16), jnp.float32))

        @pl.loop(0, rows_per_sub // zb)
        def _(kk):
            pltpu.sync_copy(
                zbuf, agg_sh.at[pl.ds(sid * rows_per_sub + kk * zb, zb)])

        plsc.subcore_barrier()

        def body(en_vmem, r_vmem):
            pltpu.sync_copy(en_vmem, agg_sh.at[r_vmem.at[0, 0]], add=True)

        pltpu.emit_pipeline(
            body,
            grid=(e // _GW,),
            in_specs=[pl.BlockSpec((_GW, h), lambda i: (i, 0)),
                      pl.BlockSpec((1, 1, _GW), lambda i: (i, 0, 0))],
            out_specs=[],
            core_axis_name=("core", "subcore"),
            dimension_semantics=(pltpu.PARALLEL,),
        )(en_hbm, r_hbm)

        plsc.subcore_barrier()

        @pl.loop(0, rows_per_sub // zb)
        def _(kk):
            pltpu.sync_copy(
                agg_sh.at[pl.ds(sid * rows_per_sub + kk * zb, zb)], zbuf)
            pltpu.sync_copy(
                zbuf,
                out_hbm.at[
                    pl.ds(cid * _OP + sid * rows_per_sub + kk * zb, zb)])

    return k(en, receivers.reshape(e // _GW, 1, _GW))


def _edge_body(gs_ref, gr_ref, attr_ref, eo_in_ref, w0e, b0, w1, b1, w2, b2,
               w3, b3, g, beta, en_ref, eo_ref):
    del eo_in_ref
    attr = attr_ref[...]
    gs32 = gs_ref[...]
    gr32 = gr_ref[...]
    f32 = lambda v: jax.lax.bitcast_convert_type(v, jnp.float32)
    lo = f32(jax.lax.shift_left(gs32, 16)) + f32(jax.lax.shift_left(gr32, 16))
    himask = jnp.int32(-65536)
    hi = f32(gs32 & himask) + f32(gr32 & himask)
    gsum = jnp.concatenate([lo, hi], axis=1)
    h = (gsum + b0[...]
         + jnp.dot(attr, w0e[...], preferred_element_type=jnp.float32,
                   precision=_PREC))
    h = jnp.maximum(h, 0.0)
    h = jnp.maximum(
        jnp.dot(h, w1[...], preferred_element_type=jnp.float32,
                precision=_PREC) + b1[...], 0.0)
    h = jnp.maximum(
        jnp.dot(h, w2[...], preferred_element_type=jnp.float32,
                precision=_PREC) + b2[...], 0.0)
    h = jnp.dot(h, w3[...], preferred_element_type=jnp.float32,
                precision=_PREC) + b3[...]
    mu = jnp.mean(h, axis=-1, keepdims=True)
    d = h - mu
    var = jnp.mean(d * d, axis=-1, keepdims=True)
    en = (d * lax.rsqrt(var + 1e-5)) * g[...] + beta[...]
    en_ref[...] = en
    eo_ref[...] = attr + en


def _edge_body0(gs_ref, gr_ref, attr_ref, w0e, b0, w1, b1, w2, b2,
                w3, b3, g, beta, en_ref, eo_ref):
    _edge_body(gs_ref, gr_ref, attr_ref, None, w0e, b0, w1, b1, w2, b2,
               w3, b3, g, beta, en_ref, eo_ref)


def _edge_mlp_chunk(gs_c, gr_c, attr, eo_buf, c, w0e, b0, w1, b1, w2, b2, w3,
                    b3, g, beta):
    e, h = attr.shape
    ec = gs_c.shape[0]
    te = 8000
    steps = ec // te
    off = c * steps
    row = lambda i: (i, 0)
    offrow = lambda i: (i + off, 0)
    whole = lambda i: (0, 0)
    wspec = pl.BlockSpec((h, h), whole)
    bspec = pl.BlockSpec((1, h), whole)
    gspecs = ([pl.BlockSpec((te, h // 2), row)] * 2
              + [pl.BlockSpec((te, h), offrow)])
    wspecs = [wspec, bspec, wspec, bspec, wspec, bspec, wspec, bspec,
              bspec, bspec]
    out_specs = [pl.BlockSpec((te, h), row), pl.BlockSpec((te, h), offrow)]
    out_shape = [jax.ShapeDtypeStruct((ec, h), jnp.float32),
                 jax.ShapeDtypeStruct((e, h), jnp.float32)]
    wargs = (w0e, b0, w1, b1, w2, b2, w3, b3, g, beta)
    if eo_buf is None:
        return pl.pallas_call(
            _edge_body0,
            grid=(steps,),
            in_specs=gspecs + wspecs,
            out_specs=out_specs,
            out_shape=out_shape,
        )(gs_c, gr_c, attr, *wargs)
    return pl.pallas_call(
        _edge_body,
        grid=(steps,),
        in_specs=gspecs + [pl.BlockSpec(memory_space=pl.ANY)] + wspecs,
        out_specs=out_specs,
        out_shape=out_shape,
        input_output_aliases={3: 1},
    )(gs_c, gr_c, attr, eo_buf, *wargs)


def _node_body(x_ref, p0, p1, p2, p3, p4, p5, p6, p7, p8, p9, wx, wa, b0,
               w1, b1, w2, b2, w3, b3, g, beta, xo_ref):
    xb = x_ref[...]
    agg = ((p0[...] + p1[...]) + (p2[...] + p3[...])
           + (p4[...] + p5[...]) + (p6[...] + p7[...])
           + (p8[...] + p9[...]))
    h = (jnp.dot(xb, wx[...], preferred_element_type=jnp.float32,
                 precision=_PREC)
         + jnp.dot(agg, wa[...], preferred_element_type=jnp.float32,
                   precision=_PREC) + b0[...])
    h = jnp.maximum(h, 0.0)
    h = jnp.maximum(
        jnp.dot(h, w1[...], preferred_element_type=jnp.float32,
                precision=_PREC) + b1[...], 0.0)
    h = jnp.maximum(
        jnp.dot(h, w2[...], preferred_element_type=jnp.float32,
                precision=_PREC) + b2[...], 0.0)
    h = jnp.dot(h, w3[...], preferred_element_type=jnp.float32,
                precision=_PREC) + b3[...]
    mu = jnp.mean(h, axis=-1, keepdims=True)
    d = h - mu
    var = jnp.mean(d * d, axis=-1, keepdims=True)
    xo_ref[...] = xb + (d * lax.rsqrt(var + 1e-5)) * g[...] + beta[...]


def _node_mlp(x, parts, wx, wa, b0, w1, b1, w2, b2, w3, b3, g, beta):
    n, h = x.shape
    tn = 2000
    row = lambda i: (i, 0)
    row1 = lambda i: (i + _OP // 2000, 0)
    whole = lambda i: (0, 0)
    wspec = pl.BlockSpec((h, h), whole)
    bspec = pl.BlockSpec((1, h), whole)
    pspecs = []
    pargs = []
    for p in parts:
        pspecs += [pl.BlockSpec((tn, h), row), pl.BlockSpec((tn, h), row1)]
        pargs += [p, p]
    return pl.pallas_call(
        _node_body,
        grid=(n // tn,),
        in_specs=[pl.BlockSpec((tn, h), row)] + pspecs
        + [wspec, wspec, bspec, wspec, bspec, wspec, bspec, wspec, bspec,
           bspec, bspec],
        out_specs=pl.BlockSpec((tn, h), row),
        out_shape=jax.ShapeDtypeStruct((n, h), jnp.float32),
    )(x, *pargs, wx, wa, b0, w1, b1, w2, b2, w3, b3, g, beta)


def kernel(x, edge_index, edge_attr, eb_W0, eb_b0, eb_W1, eb_b1, eb_W2, eb_b2,
           eb_W3, eb_b3, eb_g, eb_beta, nb_W0, nb_b0, nb_W1, nb_b1, nb_W2,
           nb_b2, nb_W3, nb_b3, nb_g, nb_beta):
    n, h = x.shape
    e = edge_attr.shape[0]
    ec = e // _K
    senders = edge_index[0].reshape(_K, ec)
    receivers = edge_index[1].reshape(_K, ec)

    r2 = lambda v: v.reshape(1, h)
    perm = jnp.concatenate([jnp.arange(0, h, 2), jnp.arange(1, h, 2)])
    eb = (eb_W0[2 * h:][:, perm], r2(eb_b0[perm]), eb_W1[perm, :], r2(eb_b1),
          eb_W2, r2(eb_b2), eb_W3, r2(eb_b3), r2(eb_g), r2(eb_beta))

    ts, tr = _make_tables(x, eb_W0[:h][:, perm], eb_W0[h:2 * h][:, perm])

    eo_buf = None
    parts = []
    for c in range(_K):
        gs_c = _sc_gather(ts, senders[c])
        gr_c = _sc_gather(tr, receivers[c])
        en_c, eo_buf = _edge_mlp_chunk(gs_c, gr_c, edge_attr, eo_buf, c, *eb)
        parts.append(_sc_scatter(en_c, receivers[c]))

    xo = _node_mlp(x, parts, nb_W0[:h], nb_W0[h:], r2(nb_b0), nb_W1,
                   r2(nb_b1), nb_W2, r2(nb_b2), nb_W3, r2(nb_b3), r2(nb_g),
                   r2(nb_beta))
    return (xo, eo_buf)


# speedup vs baseline: 1.0519x; 1.0022x over previous
import functools

import jax
import jax.numpy as jnp
from jax import lax
from jax.experimental import pallas as pl
from jax.experimental.pallas import tpu as pltpu
from jax.experimental.pallas import tpu_sc as plsc

_PREC = lax.Precision.DEFAULT
_K = 5


def _pack_bf16_pair(t):
    m = t.shape[1] // 2
    a = jax.lax.bitcast_convert_type(t[:, :m], jnp.int32)
    b = jax.lax.bitcast_convert_type(t[:, m:], jnp.int32)
    a = jax.lax.shift_right_logical(a + 0x8000, 16)
    b = (b + 0x8000) & jnp.int32(-65536)
    return a | b


def _tables_body(x_ref, ws_ref, wr_ref, ts_ref, tr_ref):
    xb = x_ref[...]
    ts = jnp.dot(xb, ws_ref[...], preferred_element_type=jnp.float32,
                 precision=_PREC)
    tr = jnp.dot(xb, wr_ref[...], preferred_element_type=jnp.float32,
                 precision=_PREC)
    ts_ref[...] = _pack_bf16_pair(ts)
    tr_ref[...] = _pack_bf16_pair(tr)


def _make_tables(x, ws, wr):
    n, h = x.shape
    tb = 2000
    return pl.pallas_call(
        _tables_body,
        grid=(n // tb,),
        in_specs=[
            pl.BlockSpec((tb, h), lambda i: (i, 0)),
            pl.BlockSpec((h, h), lambda i: (0, 0)),
            pl.BlockSpec((h, h), lambda i: (0, 0)),
        ],
        out_specs=[
            pl.BlockSpec((tb, h // 2), lambda i: (i, 0)),
            pl.BlockSpec((tb, h // 2), lambda i: (i, 0)),
        ],
        out_shape=[jax.ShapeDtypeStruct((n, h // 2), jnp.int32)] * 2,
    )(x, ws, wr)


_GW = 80


def _sc_gather2(ts, tr, sidx, ridx):
    n, h = ts.shape
    e = sidx.shape[0]
    mesh = plsc.VectorSubcoreMesh(core_axis_name="core",
                                  subcore_axis_name="subcore")

    @functools.partial(
        pl.kernel,
        out_type=[jax.ShapeDtypeStruct((e, h), ts.dtype)] * 2,
        mesh=mesh,
        compiler_params=pltpu.CompilerParams(use_tc_tiling_on_sc=False),
    )
    def k(ts_hbm, tr_hbm, s_hbm, r_hbm, gs_hbm, gr_hbm):
        def body(s_vmem, r_vmem, gs_vmem, gr_vmem):
            pltpu.sync_copy(ts_hbm.at[s_vmem.at[0, 0]], gs_vmem)
            pltpu.sync_copy(tr_hbm.at[r_vmem.at[0, 0]], gr_vmem)

        pltpu.emit_pipeline(
            body,
            grid=(e // _GW,),
            in_specs=[pl.BlockSpec((1, 1, _GW), lambda i: (i, 0, 0))] * 2,
            out_specs=[pl.BlockSpec((_GW, h), lambda i: (i, 0))] * 2,
            core_axis_name=("core", "subcore"),
            dimension_semantics=(pltpu.PARALLEL,),
        )(s_hbm, r_hbm, gs_hbm, gr_hbm)

    return k(ts, tr, sidx.reshape(e // _GW, 1, _GW),
             ridx.reshape(e // _GW, 1, _GW))


_NPAD = 10240
_OP = 12000


def _sc_scatter(en, receivers):
    e, h = en.shape
    n_sub = 16
    rows_per_sub = _NPAD // n_sub
    zb = 128
    mesh = plsc.VectorSubcoreMesh(core_axis_name="core",
                                  subcore_axis_name="subcore")

    @functools.partial(
        pl.kernel,
        out_type=jax.ShapeDtypeStruct((2 * _OP, h), jnp.float32),
        mesh=mesh,
        scratch_types=[
            pltpu.VMEM((zb, h), jnp.float32),
            pltpu.VMEM_SHARED((_NPAD, h), jnp.float32),
        ],
    )
    def k(en_hbm, r_hbm, out_hbm, zbuf, agg_sh):
        cid = lax.axis_index("core")
        sid = lax.axis_index("subcore")

        @pl.loop(0, zb)
        def _(rr):
            for j in range(h // 16):
                zbuf.at[pl.ds(rr, 1), pl.ds(j * 16, 16)][...] = (
                    jnp.zeros((1, 16), jnp.float32))

        @pl.loop(0, rows_per_sub // zb)
        def _(kk):
            pltpu.sync_copy(
                zbuf, agg_sh.at[pl.ds(sid * rows_per_sub + kk * zb, zb)])

        plsc.subcore_barrier()

        def body(en_vmem, r_vmem):
            pltpu.sync_copy(en_vmem, agg_sh.at[r_vmem.at[0, 0]], add=True)

        pltpu.emit_pipeline(
            body,
            grid=(e // _GW,),
            in_specs=[pl.BlockSpec((_GW, h), lambda i: (i, 0)),
                      pl.BlockSpec((1, 1, _GW), lambda i: (i, 0, 0))],
            out_specs=[],
            core_axis_name=("core", "subcore"),
            dimension_semantics=(pltpu.PARALLEL,),
        )(en_hbm, r_hbm)

        plsc.subcore_barrier()

        @pl.loop(0, rows_per_sub // zb)
        def _(kk):
            pltpu.sync_copy(
                agg_sh.at[pl.ds(sid * rows_per_sub + kk * zb, zb)], zbuf)
            pltpu.sync_copy(
                zbuf,
                out_hbm.at[
                    pl.ds(cid * _OP + sid * rows_per_sub + kk * zb, zb)])

    return k(en, receivers.reshape(e // _GW, 1, _GW))


def _edge_body(gs_ref, gr_ref, attr_ref, eo_in_ref, w0e, b0, w1, b1, w2, b2,
               w3, b3, g, beta, en_ref, eo_ref):
    del eo_in_ref
    attr = attr_ref[...]
    gs32 = gs_ref[...]
    gr32 = gr_ref[...]
    f32 = lambda v: jax.lax.bitcast_convert_type(v, jnp.float32)
    lo = f32(jax.lax.shift_left(gs32, 16)) + f32(jax.lax.shift_left(gr32, 16))
    himask = jnp.int32(-65536)
    hi = f32(gs32 & himask) + f32(gr32 & himask)
    gsum = jnp.concatenate([lo, hi], axis=1)
    h = (gsum + b0[...]
         + jnp.dot(attr, w0e[...], preferred_element_type=jnp.float32,
                   precision=_PREC))
    h = jnp.maximum(h, 0.0)
    h = jnp.maximum(
        jnp.dot(h, w1[...], preferred_element_type=jnp.float32,
                precision=_PREC) + b1[...], 0.0)
    h = jnp.maximum(
        jnp.dot(h, w2[...], preferred_element_type=jnp.float32,
                precision=_PREC) + b2[...], 0.0)
    h = jnp.dot(h, w3[...], preferred_element_type=jnp.float32,
                precision=_PREC) + b3[...]
    mu = jnp.mean(h, axis=-1, keepdims=True)
    d = h - mu
    var = jnp.mean(d * d, axis=-1, keepdims=True)
    en = (d * lax.rsqrt(var + 1e-5)) * g[...] + beta[...]
    en_ref[...] = en
    eo_ref[...] = attr + en


def _edge_body0(gs_ref, gr_ref, attr_ref, w0e, b0, w1, b1, w2, b2,
                w3, b3, g, beta, en_ref, eo_ref):
    _edge_body(gs_ref, gr_ref, attr_ref, None, w0e, b0, w1, b1, w2, b2,
               w3, b3, g, beta, en_ref, eo_ref)


def _edge_mlp_chunk(gs_c, gr_c, attr, eo_buf, c, w0e, b0, w1, b1, w2, b2, w3,
                    b3, g, beta):
    e, h = attr.shape
    ec = gs_c.shape[0]
    te = 8000
    steps = ec // te
    off = c * steps
    row = lambda i: (i, 0)
    offrow = lambda i: (i + off, 0)
    whole = lambda i: (0, 0)
    wspec = pl.BlockSpec((h, h), whole)
    bspec = pl.BlockSpec((1, h), whole)
    gspecs = ([pl.BlockSpec((te, h // 2), row)] * 2
              + [pl.BlockSpec((te, h), offrow)])
    wspecs = [wspec, bspec, wspec, bspec, wspec, bspec, wspec, bspec,
              bspec, bspec]
    out_specs = [pl.BlockSpec((te, h), row), pl.BlockSpec((te, h), offrow)]
    out_shape = [jax.ShapeDtypeStruct((ec, h), jnp.float32),
                 jax.ShapeDtypeStruct((e, h), jnp.float32)]
    wargs = (w0e, b0, w1, b1, w2, b2, w3, b3, g, beta)
    if eo_buf is None:
        return pl.pallas_call(
            _edge_body0,
            grid=(steps,),
            in_specs=gspecs + wspecs,
            out_specs=out_specs,
            out_shape=out_shape,
        )(gs_c, gr_c, attr, *wargs)
    return pl.pallas_call(
        _edge_body,
        grid=(steps,),
        in_specs=gspecs + [pl.BlockSpec(memory_space=pl.ANY)] + wspecs,
        out_specs=out_specs,
        out_shape=out_shape,
        input_output_aliases={3: 1},
    )(gs_c, gr_c, attr, eo_buf, *wargs)


def _node_body(x_ref, p0, p1, p2, p3, p4, p5, p6, p7, p8, p9, wx, wa, b0,
               w1, b1, w2, b2, w3, b3, g, beta, xo_ref):
    xb = x_ref[...]
    agg = ((p0[...] + p1[...]) + (p2[...] + p3[...])
           + (p4[...] + p5[...]) + (p6[...] + p7[...])
           + (p8[...] + p9[...]))
    h = (jnp.dot(xb, wx[...], preferred_element_type=jnp.float32,
                 precision=_PREC)
         + jnp.dot(agg, wa[...], preferred_element_type=jnp.float32,
                   precision=_PREC) + b0[...])
    h = jnp.maximum(h, 0.0)
    h = jnp.maximum(
        jnp.dot(h, w1[...], preferred_element_type=jnp.float32,
                precision=_PREC) + b1[...], 0.0)
    h = jnp.maximum(
        jnp.dot(h, w2[...], preferred_element_type=jnp.float32,
                precision=_PREC) + b2[...], 0.0)
    h = jnp.dot(h, w3[...], preferred_element_type=jnp.float32,
                precision=_PREC) + b3[...]
    mu = jnp.mean(h, axis=-1, keepdims=True)
    d = h - mu
    var = jnp.mean(d * d, axis=-1, keepdims=True)
    xo_ref[...] = xb + (d * lax.rsqrt(var + 1e-5)) * g[...] + beta[...]


def _node_mlp(x, parts, wx, wa, b0, w1, b1, w2, b2, w3, b3, g, beta):
    n, h = x.shape
    tn = 2000
    row = lambda i: (i, 0)
    row1 = lambda i: (i + _OP // 2000, 0)
    whole = lambda i: (0, 0)
    wspec = pl.BlockSpec((h, h), whole)
    bspec = pl.BlockSpec((1, h), whole)
    pspecs = []
    pargs = []
    for p in parts:
        pspecs += [pl.BlockSpec((tn, h), row), pl.BlockSpec((tn, h), row1)]
        pargs += [p, p]
    return pl.pallas_call(
        _node_body,
        grid=(n // tn,),
        in_specs=[pl.BlockSpec((tn, h), row)] + pspecs
        + [wspec, wspec, bspec, wspec, bspec, wspec, bspec, wspec, bspec,
           bspec, bspec],
        out_specs=pl.BlockSpec((tn, h), row),
        out_shape=jax.ShapeDtypeStruct((n, h), jnp.float32),
    )(x, *pargs, wx, wa, b0, w1, b1, w2, b2, w3, b3, g, beta)


def kernel(x, edge_index, edge_attr, eb_W0, eb_b0, eb_W1, eb_b1, eb_W2, eb_b2,
           eb_W3, eb_b3, eb_g, eb_beta, nb_W0, nb_b0, nb_W1, nb_b1, nb_W2,
           nb_b2, nb_W3, nb_b3, nb_g, nb_beta):
    n, h = x.shape
    e = edge_attr.shape[0]
    ec = e // _K
    senders = edge_index[0].reshape(_K, ec)
    receivers = edge_index[1].reshape(_K, ec)

    r2 = lambda v: v.reshape(1, h)
    perm = jnp.concatenate([jnp.arange(0, h, 2), jnp.arange(1, h, 2)])
    eb = (eb_W0[2 * h:][:, perm], r2(eb_b0[perm]), eb_W1[perm, :], r2(eb_b1),
          eb_W2, r2(eb_b2), eb_W3, r2(eb_b3), r2(eb_g), r2(eb_beta))

    ts, tr = _make_tables(x, eb_W0[:h][:, perm], eb_W0[h:2 * h][:, perm])

    eo_buf = None
    parts = []
    for c in range(_K):
        gs_c, gr_c = _sc_gather2(ts, tr, senders[c], receivers[c])
        en_c, eo_buf = _edge_mlp_chunk(gs_c, gr_c, edge_attr, eo_buf, c, *eb)
        parts.append(_sc_scatter(en_c, receivers[c]))

    xo = _node_mlp(x, parts, nb_W0[:h], nb_W0[h:], r2(nb_b0), nb_W1,
                   r2(nb_b1), nb_W2, r2(nb_b2), nb_W3, r2(nb_b3), r2(nb_g),
                   r2(nb_beta))
    return (xo, eo_buf)
